# baseline (speedup 1.0000x reference)
import functools

import jax
import jax.numpy as jnp
from jax import lax
from jax.experimental import pallas as pl
from jax.experimental.pallas import tpu as pltpu
from jax.experimental.pallas import tpu_sc as plsc

NUM_LANES = 16
NUM_CORES = 2
NUM_SUBCORES = 16
NUM_WORKERS = NUM_CORES * NUM_SUBCORES

BATCH = 16384
EMBED_DIM = 32
NROWS = 1000000
PADW = 128

WINE = 768
NWIN = 1303
LASTBASE = 999296
WPW = NWIN // NUM_WORKERS
WEXTRA = NWIN - WPW * NUM_WORKERS
HCAP = 1024
WCAP = 48
SCHUNK = 2048
STAG = BATCH + WCAP

BPW = BATCH // NUM_WORKERS
HB = BPW // 2
NBLK = BPW // NUM_LANES


def _win_base(w):
    return pl.multiple_of(jnp.minimum(w * WINE, LASTBASE), 128)


def _win_of(e):
    return jnp.minimum(e // WINE, NWIN - 1)


def _compact_round(ids_v, he_v, hp_v, w0, w1, pos0, off):
    lanei = lax.iota(jnp.int32, 16)
    UNROLL = 4

    def body(v4, o):
        es, ranks, pcs, masks = [], [], [], []
        for k in range(UNROLL):
            v = v4 * UNROLL + k
            e = ids_v[pl.ds(v * NUM_LANES, NUM_LANES)]
            win = _win_of(e)
            m = (win >= w0) & (win < w1)
            es.append(e)
            masks.append(m)
            ranks.append(plsc.cumsum(m.astype(jnp.int32)) - 1)
            pcs.append(plsc.all_reduce_population_count(m)[0])
        for k in range(UNROLL):
            v = v4 * UNROLL + k
            slots = o + ranks[k]
            plsc.store_scatter(he_v, [slots], es[k], mask=masks[k])
            pos = pos0 + v * NUM_LANES + lanei
            plsc.store_scatter(hp_v, [slots], pos, mask=masks[k])
            o = o + pcs[k]
        return o

    return lax.fori_loop(0, SCHUNK // NUM_LANES // UNROLL, body, off)


def _wave_hits(he_v, hp_v, cnt, wtarget, wcol_v, wpos_v, eb):
    lanei = lax.iota(jnp.int32, 16)
    for k in range(WCAP // NUM_LANES):
        wpos_v[pl.ds(k * NUM_LANES, NUM_LANES)] = (
            BATCH + k * NUM_LANES + lanei)

    UNROLL = 4

    def body(hv4, woff):
        es, ps, ranks, pcs, masks = [], [], [], [], []
        for k in range(UNROLL):
            base = (hv4 * UNROLL + k) * NUM_LANES
            e = he_v[pl.ds(pl.multiple_of(base, NUM_LANES), NUM_LANES)]
            p = hp_v[pl.ds(pl.multiple_of(base, NUM_LANES), NUM_LANES)]
            m = (_win_of(e) == wtarget) & (base + lanei < cnt)
            es.append(e)
            ps.append(p)
            masks.append(m)
            ranks.append(plsc.cumsum(m.astype(jnp.int32)) - 1)
            pcs.append(plsc.all_reduce_population_count(m)[0])
        for k in range(UNROLL):
            slots = woff + ranks[k]
            plsc.store_scatter(wcol_v, [slots], es[k] - eb, mask=masks[k])
            plsc.store_scatter(wpos_v, [slots], ps[k], mask=masks[k])
            woff = woff + pcs[k]
        return woff

    nhv4 = lax.shift_right_logical(cnt + UNROLL * NUM_LANES - 1, 6)
    return lax.fori_loop(0, nhv4, body, jnp.int32(0))


def _gather_rows(win_v, wcol_v, wcnt, row_v):
    lanei = lax.iota(jnp.int32, 16)

    def body(g, _):
        base = g * NUM_LANES
        col = wcol_v[pl.ds(pl.multiple_of(base, NUM_LANES), NUM_LANES)]
        valid = base + lanei < wcnt
        slot = base + lanei
        for d in range(EMBED_DIM):
            dvec = jnp.full((NUM_LANES,), d, jnp.int32)
            vals = plsc.load_gather(win_v, [dvec, col], mask=valid)
            plsc.store_scatter(row_v, [slot, dvec], vals, mask=valid)
        return 0

    ngv = lax.shift_right_logical(wcnt + NUM_LANES - 1, 4)
    lax.fori_loop(0, ngv, body, 0)


def _scan_body(uids_hbm, iids_hbm, utab_hbm, itab_hbm,
               ustag_hbm, istag_hbm,
               ids_v, uhe_v, uhp_v, ihe_v, ihp_v,
               uwin0, uwin1, iwin0, iwin1,
               ucol_v, upos0, upos1, icol_v, ipos0, ipos1,
               urow0, urow1, irow0, irow1,
               uws0, uws1, iws0, iws1, usc0, usc1, isc0, isc1):
    wid = lax.axis_index("s") * NUM_CORES + lax.axis_index("c")
    w0 = wid * WPW + jnp.minimum(wid, WEXTRA)
    nw = WPW + (wid < WEXTRA).astype(jnp.int32)

    uwins = (uwin0, uwin1)
    iwins = (iwin0, iwin1)
    uwsems = (uws0, uws1)
    iwsems = (iws0, iws1)
    uposs = (upos0, upos1)
    iposs = (ipos0, ipos1)
    urows = (urow0, urow1)
    irows = (irow0, irow1)
    uscs = (usc0, usc1)
    iscs = (isc0, isc1)

    def fire(t, b):
        eb = _win_base(w0 + t)
        for g in range(EMBED_DIM // 8):
            rs = pl.ds(8 * g, 8)
            pltpu.async_copy(utab_hbm.at[rs, pl.ds(eb, WINE)],
                             uwins[b].at[rs], uwsems[b])
            pltpu.async_copy(itab_hbm.at[rs, pl.ds(eb, WINE)],
                             iwins[b].at[rs], iwsems[b])

    def wait_win(t, b):
        eb = _win_base(w0 + t)
        for g in range(EMBED_DIM // 8):
            rs = pl.ds(8 * g, 8)
            pltpu.make_async_copy(utab_hbm.at[rs, pl.ds(eb, WINE)],
                                  uwins[b].at[rs], uwsems[b]).wait()
            pltpu.make_async_copy(itab_hbm.at[rs, pl.ds(eb, WINE)],
                                  iwins[b].at[rs], iwsems[b]).wait()

    fire(0, 0)

    @pl.when(nw > 1)
    def _():
        fire(1, 1)

    def compact(ids_hbm, he_v, hp_v):
        off = jnp.int32(0)
        for r in range(BATCH // SCHUNK):
            pltpu.sync_copy(ids_hbm.at[pl.ds(r * SCHUNK, SCHUNK)], ids_v)
            off = _compact_round(ids_v, he_v, hp_v, w0, w0 + nw,
                                 r * SCHUNK, off)
        return off

    ucnt = compact(uids_hbm, uhe_v, uhp_v)
    icnt = compact(iids_hbm, ihe_v, ihp_v)

    def step(t, b):
        eb = _win_base(w0 + t)
        wait_win(t, b)
        @pl.when(t >= 2)
        def _():
            pltpu.make_async_copy(urows[b], ustag_hbm.at[uposs[b]],
                                  uscs[b]).wait()
            pltpu.make_async_copy(irows[b], istag_hbm.at[iposs[b]],
                                  iscs[b]).wait()

        lanei2 = lax.iota(jnp.int32, 16)
        for k in range(WCAP // NUM_LANES):
            uposs[b][pl.ds(k * NUM_LANES, NUM_LANES)] = (
                BATCH + k * NUM_LANES + lanei2)
            iposs[b][pl.ds(k * NUM_LANES, NUM_LANES)] = (
                BATCH + k * NUM_LANES + lanei2)
        pltpu.async_copy(urows[b], ustag_hbm.at[uposs[b]], uscs[b])
        pltpu.async_copy(irows[b], istag_hbm.at[iposs[b]], iscs[b])

        @pl.when(t + 2 < nw)
        def _():
            fire(t + 2, b)

    def outer(t2, _):
        for b in range(2):
            t = t2 * 2 + b

            @pl.when(t < nw)
            def _():
                step(t, b)
        return 0

    lax.fori_loop(0, (WPW + 2) // 2, outer, 0)

    def tail(t2, _):
        for b in range(2):
            t = t2 * 2 + b

            @pl.when((t < nw) & (t + 2 >= nw))
            def _():
                pltpu.make_async_copy(urows[b], ustag_hbm.at[uposs[b]],
                                      uscs[b]).wait()
                pltpu.make_async_copy(irows[b], istag_hbm.at[iposs[b]],
                                      iscs[b]).wait()
        return 0

    lax.fori_loop(0, (WPW + 2) // 2, tail, 0)


def _dist_body(ustag_hbm, istag_hbm, out_hbm, ubuf_v, ibuf_v, out_v,
               usem, isem):
    wid = lax.axis_index("s") * NUM_CORES + lax.axis_index("c")
    base = wid * BPW

    lane = lax.iota(jnp.int32, 16)
    zero = jnp.zeros((NUM_LANES,), jnp.float32)
    half = EMBED_DIM // 2

    def load_half(h):
        off = pl.multiple_of(base + h * HB, HB)
        cu = pltpu.async_copy(ustag_hbm.at[pl.ds(off, HB)], ubuf_v, usem)
        ci = pltpu.async_copy(istag_hbm.at[pl.ds(off, HB)], ibuf_v, isem)
        cu.wait()
        ci.wait()

    def blk(blk_i, _):
        h = blk_i // (HB // NUM_LANES)

        @pl.when((blk_i % (HB // NUM_LANES)) == 0)
        def _():
            load_half(h)

        base_row = pl.multiple_of(
            (blk_i % (HB // NUM_LANES)) * NUM_LANES, NUM_LANES)
        acc_p, acc_u, acc_i = zero, zero, zero
        for r in range(NUM_LANES):
            row = base_row + r
            u0 = ubuf_v[row, pl.ds(0, half)]
            u1 = ubuf_v[row, pl.ds(half, half)]
            i0 = ibuf_v[row, pl.ds(0, half)]
            i1 = ibuf_v[row, pl.ds(half, half)]
            p0 = u0 * i0
            p1 = u1 * i1
            sp = jnp.sum(p0 * p0 + p1 * p1)
            su = jnp.sum(u0 * u0 + u1 * u1)
            si = jnp.sum(i0 * i0 + i1 * i1)
            m = lane == r
            acc_p = jnp.where(m, sp, acc_p)
            acc_u = jnp.where(m, su, acc_u)
            acc_i = jnp.where(m, si, acc_i)
        denom = jnp.maximum(acc_u, 1.0) * jnp.maximum(acc_i, 1.0)
        out_v[pl.ds(pl.multiple_of(h * HB, HB) + base_row, NUM_LANES)] = (
            -(acc_p / denom))
        return 0

    lax.fori_loop(0, NBLK, blk, 0)
    pltpu.sync_copy(out_v, out_hbm.at[pl.ds(base, BPW)])


_params = pltpu.CompilerParams(needs_layout_passes=False,
                               use_tc_tiling_on_sc=True)


@jax.jit
def _cml(user_ids, item_ids, user_table, item_table):
    mesh = plsc.VectorSubcoreMesh(core_axis_name="c", subcore_axis_name="s")
    scan = functools.partial(
        pl.kernel,
        out_type=(jax.ShapeDtypeStruct((STAG, PADW), jnp.float32),
                  jax.ShapeDtypeStruct((STAG, PADW), jnp.float32)),
        mesh=mesh,
        compiler_params=_params,
        scratch_types=[
            pltpu.VMEM((SCHUNK,), jnp.int32),
            pltpu.VMEM((HCAP,), jnp.int32),
            pltpu.VMEM((HCAP,), jnp.int32),
            pltpu.VMEM((HCAP,), jnp.int32),
            pltpu.VMEM((HCAP,), jnp.int32),
            pltpu.VMEM((EMBED_DIM, WINE), jnp.float32),
            pltpu.VMEM((EMBED_DIM, WINE), jnp.float32),
            pltpu.VMEM((EMBED_DIM, WINE), jnp.float32),
            pltpu.VMEM((EMBED_DIM, WINE), jnp.float32),
            pltpu.VMEM((WCAP,), jnp.int32),
            pltpu.VMEM((WCAP,), jnp.int32),
            pltpu.VMEM((WCAP,), jnp.int32),
            pltpu.VMEM((WCAP,), jnp.int32),
            pltpu.VMEM((WCAP,), jnp.int32),
            pltpu.VMEM((WCAP,), jnp.int32),
            pltpu.VMEM((WCAP, PADW), jnp.float32),
            pltpu.VMEM((WCAP, PADW), jnp.float32),
            pltpu.VMEM((WCAP, PADW), jnp.float32),
            pltpu.VMEM((WCAP, PADW), jnp.float32),
        ] + [pltpu.SemaphoreType.DMA] * 8,
    )(_scan_body)
    ustag, istag = scan(user_ids, item_ids, user_table.T, item_table.T)

    dist = functools.partial(
        pl.kernel,
        out_type=jax.ShapeDtypeStruct((BATCH,), jnp.float32),
        mesh=mesh,
        compiler_params=_params,
        scratch_types=[
            pltpu.VMEM((HB, PADW), jnp.float32),
            pltpu.VMEM((HB, PADW), jnp.float32),
            pltpu.VMEM((BPW,), jnp.float32),
            pltpu.SemaphoreType.DMA,
            pltpu.SemaphoreType.DMA,
        ],
    )(_dist_body)
    return dist(ustag, istag)


def kernel(user_ids, item_ids, user_table, item_table):
    return _cml(user_ids, item_ids, user_table, item_table)


# speedup vs baseline: 1.3685x; 1.3685x over previous
import functools

import jax
import jax.numpy as jnp
from jax import lax
from jax.experimental import pallas as pl
from jax.experimental.pallas import tpu as pltpu
from jax.experimental.pallas import tpu_sc as plsc

NUM_LANES = 16
NUM_CORES = 2
NUM_SUBCORES = 16
NUM_WORKERS = NUM_CORES * NUM_SUBCORES

BATCH = 16384
EMBED_DIM = 32
NROWS = 1000000
PADW = 128

WINE = 8192
SLICE = WINE // NUM_SUBCORES
NWIN = 123
LASTBASE = 991872
W0SC = 62
HCAP = 1024
WCAP = 32
SCHUNK = 2048
STAG = BATCH + WCAP

BPW = BATCH // NUM_WORKERS
HB = BPW // 2
NBLK = BPW // NUM_LANES


def _win_base(w):
    return pl.multiple_of(jnp.minimum(w * WINE, LASTBASE), 128)


def _compact_round(ids_v, he_v, hp_v, w0, w1, sid, pos0, off):
    lanei = lax.iota(jnp.int32, 16)
    UNROLL = 4

    def body(v4, o):
        es, ranks, pcs, masks = [], [], [], []
        for k in range(UNROLL):
            v = v4 * UNROLL + k
            e = ids_v[pl.ds(v * NUM_LANES, NUM_LANES)]
            win = jnp.minimum(lax.shift_right_logical(e, 13), NWIN - 1)
            wbase = jnp.minimum(win * WINE, LASTBASE)
            sub = lax.shift_right_logical(e - wbase, 9)
            m = (win >= w0) & (win < w1) & (sub == sid)
            es.append(e)
            masks.append(m)
            ranks.append(plsc.cumsum(m.astype(jnp.int32)) - 1)
            pcs.append(plsc.all_reduce_population_count(m)[0])
        for k in range(UNROLL):
            v = v4 * UNROLL + k
            slots = o + ranks[k]
            plsc.store_scatter(he_v, [slots], es[k], mask=masks[k])
            pos = pos0 + v * NUM_LANES + lanei
            plsc.store_scatter(hp_v, [slots], pos, mask=masks[k])
            o = o + pcs[k]
        return o

    return lax.fori_loop(0, SCHUNK // NUM_LANES // UNROLL, body, off)


def _wave_hits(he_v, hp_v, cnt, wtarget, wcol_v, wpos_v, eb):
    lanei = lax.iota(jnp.int32, 16)
    for k in range(WCAP // NUM_LANES):
        wpos_v[pl.ds(k * NUM_LANES, NUM_LANES)] = (
            BATCH + k * NUM_LANES + lanei)

    UNROLL = 4

    def body(hv4, woff):
        es, ps, ranks, pcs, masks = [], [], [], [], []
        for k in range(UNROLL):
            base = (hv4 * UNROLL + k) * NUM_LANES
            e = he_v[pl.ds(pl.multiple_of(base, NUM_LANES), NUM_LANES)]
            p = hp_v[pl.ds(pl.multiple_of(base, NUM_LANES), NUM_LANES)]
            win = jnp.minimum(lax.shift_right_logical(e, 13), NWIN - 1)
            m = (win == wtarget) & (base + lanei < cnt)
            es.append(e)
            ps.append(p)
            masks.append(m)
            ranks.append(plsc.cumsum(m.astype(jnp.int32)) - 1)
            pcs.append(plsc.all_reduce_population_count(m)[0])
        for k in range(UNROLL):
            slots = woff + ranks[k]
            plsc.store_scatter(wcol_v, [slots], es[k] - eb, mask=masks[k])
            plsc.store_scatter(wpos_v, [slots], ps[k], mask=masks[k])
            woff = woff + pcs[k]
        return woff

    nhv4 = lax.shift_right_logical(cnt + UNROLL * NUM_LANES - 1, 6)
    return lax.fori_loop(0, nhv4, body, jnp.int32(0))


def _gather_rows(win_v, wcol_v, wcnt, row_v):
    lanei = lax.iota(jnp.int32, 16)

    def body(g, _):
        base = g * NUM_LANES
        col = wcol_v[pl.ds(pl.multiple_of(base, NUM_LANES), NUM_LANES)]
        valid = base + lanei < wcnt
        slot = base + lanei
        for d in range(EMBED_DIM):
            dvec = jnp.full((NUM_LANES,), d, jnp.int32)
            vals = plsc.load_gather(win_v, [dvec, col], mask=valid)
            plsc.store_scatter(row_v, [slot, dvec], vals, mask=valid)
        return 0

    ngv = lax.shift_right_logical(wcnt + NUM_LANES - 1, 4)
    lax.fori_loop(0, ngv, body, 0)


def _scan_body(uids_hbm, iids_hbm, utab_hbm, itab_hbm,
               ustag_hbm, istag_hbm,
               ids_v, uhe_v, uhp_v, ihe_v, ihp_v,
               uspm0, uspm1, ispm0, ispm1,
               uslice_v, islice_v,
               ucol_v, upos0, upos1, icol_v, ipos0, ipos1,
               urow0, urow1, irow0, irow1,
               uws0, uws1, iws0, iws1, usc0, usc1, isc0, isc1):
    cid = lax.axis_index("c")
    sid = lax.axis_index("s")
    w0 = cid * W0SC
    nw = (W0SC + 1) - cid

    uspms = (uspm0, uspm1)
    ispms = (ispm0, ispm1)
    uwsems = (uws0, uws1)
    iwsems = (iws0, iws1)
    uposs = (upos0, upos1)
    iposs = (ipos0, ipos1)
    urows = (urow0, urow1)
    irows = (irow0, irow1)
    uscs = (usc0, usc1)
    iscs = (isc0, isc1)

    def fire(t, b):
        eb = _win_base(w0 + t)
        for g in range(EMBED_DIM // 8):
            rs = pl.ds(8 * g, 8)
            pltpu.async_copy(utab_hbm.at[rs, pl.ds(eb, WINE)],
                             uspms[b].at[rs], uwsems[b])
            pltpu.async_copy(itab_hbm.at[rs, pl.ds(eb, WINE)],
                             ispms[b].at[rs], iwsems[b])

    def wait_win(t, b):
        eb = _win_base(w0 + t)
        for g in range(EMBED_DIM // 8):
            rs = pl.ds(8 * g, 8)
            pltpu.make_async_copy(utab_hbm.at[rs, pl.ds(eb, WINE)],
                                  uspms[b].at[rs], uwsems[b]).wait()
            pltpu.make_async_copy(itab_hbm.at[rs, pl.ds(eb, WINE)],
                                  ispms[b].at[rs], iwsems[b]).wait()

    @pl.when(sid == 0)
    def _():
        fire(0, 0)
        fire(1, 1)

    def compact(ids_hbm, he_v, hp_v):
        off = jnp.int32(0)
        for r in range(BATCH // SCHUNK):
            pltpu.sync_copy(ids_hbm.at[pl.ds(r * SCHUNK, SCHUNK)], ids_v)
            off = _compact_round(ids_v, he_v, hp_v, w0, w0 + nw, sid,
                                 r * SCHUNK, off)
        return off

    ucnt = compact(uids_hbm, uhe_v, uhp_v)
    icnt = compact(iids_hbm, ihe_v, ihp_v)

    def step(t, b):
        eb = _win_base(w0 + t) + sid * SLICE

        @pl.when(sid == 0)
        def _():
            wait_win(t, b)

        plsc.subcore_barrier()
        soff = pl.multiple_of(sid * SLICE, SLICE)
        pltpu.sync_copy(uspms[b].at[:, pl.ds(soff, SLICE)], uslice_v)
        pltpu.sync_copy(ispms[b].at[:, pl.ds(soff, SLICE)], islice_v)
        plsc.subcore_barrier()

        @pl.when((sid == 0) & (t + 2 < nw))
        def _():
            fire(t + 2, b)

        @pl.when(t >= 2)
        def _():
            pltpu.make_async_copy(urows[b], ustag_hbm.at[uposs[b]],
                                  uscs[b]).wait()
            pltpu.make_async_copy(irows[b], istag_hbm.at[iposs[b]],
                                  iscs[b]).wait()

        uw = _wave_hits(uhe_v, uhp_v, ucnt, w0 + t, ucol_v, uposs[b], eb)
        iw = _wave_hits(ihe_v, ihp_v, icnt, w0 + t, icol_v, iposs[b], eb)
        _gather_rows(uslice_v, ucol_v, uw, urows[b])
        _gather_rows(islice_v, icol_v, iw, irows[b])
        pltpu.async_copy(urows[b], ustag_hbm.at[uposs[b]], uscs[b])
        pltpu.async_copy(irows[b], istag_hbm.at[iposs[b]], iscs[b])

    def outer(t2, _):
        for b in range(2):
            t = t2 * 2 + b

            @pl.when(t < nw)
            def _():
                step(t, b)
        return 0

    lax.fori_loop(0, (W0SC + 2) // 2, outer, 0)

    def tail(t2, _):
        for b in range(2):
            t = t2 * 2 + b

            @pl.when((t < nw) & (t + 2 >= nw))
            def _():
                pltpu.make_async_copy(urows[b], ustag_hbm.at[uposs[b]],
                                      uscs[b]).wait()
                pltpu.make_async_copy(irows[b], istag_hbm.at[iposs[b]],
                                      iscs[b]).wait()
        return 0

    lax.fori_loop(0, (W0SC + 2) // 2, tail, 0)


def _dist_body(ustag_hbm, istag_hbm, out_hbm, ubuf_v, ibuf_v, out_v,
               usem, isem):
    wid = lax.axis_index("s") * NUM_CORES + lax.axis_index("c")
    base = wid * BPW

    lane = lax.iota(jnp.int32, 16)
    zero = jnp.zeros((NUM_LANES,), jnp.float32)
    half = EMBED_DIM // 2

    def load_half(h):
        off = pl.multiple_of(base + h * HB, HB)
        cu = pltpu.async_copy(ustag_hbm.at[pl.ds(off, HB)], ubuf_v, usem)
        ci = pltpu.async_copy(istag_hbm.at[pl.ds(off, HB)], ibuf_v, isem)
        cu.wait()
        ci.wait()

    def blk(blk_i, _):
        h = blk_i // (HB // NUM_LANES)

        @pl.when((blk_i % (HB // NUM_LANES)) == 0)
        def _():
            load_half(h)

        base_row = pl.multiple_of(
            (blk_i % (HB // NUM_LANES)) * NUM_LANES, NUM_LANES)
        acc_p, acc_u, acc_i = zero, zero, zero
        for r in range(NUM_LANES):
            row = base_row + r
            u0 = ubuf_v[row, pl.ds(0, half)]
            u1 = ubuf_v[row, pl.ds(half, half)]
            i0 = ibuf_v[row, pl.ds(0, half)]
            i1 = ibuf_v[row, pl.ds(half, half)]
            p0 = u0 * i0
            p1 = u1 * i1
            sp = jnp.sum(p0 * p0 + p1 * p1)
            su = jnp.sum(u0 * u0 + u1 * u1)
            si = jnp.sum(i0 * i0 + i1 * i1)
            m = lane == r
            acc_p = jnp.where(m, sp, acc_p)
            acc_u = jnp.where(m, su, acc_u)
            acc_i = jnp.where(m, si, acc_i)
        denom = jnp.maximum(acc_u, 1.0) * jnp.maximum(acc_i, 1.0)
        out_v[pl.ds(pl.multiple_of(h * HB, HB) + base_row, NUM_LANES)] = (
            -(acc_p / denom))
        return 0

    lax.fori_loop(0, NBLK, blk, 0)
    pltpu.sync_copy(out_v, out_hbm.at[pl.ds(base, BPW)])


_params = pltpu.CompilerParams(needs_layout_passes=False,
                               use_tc_tiling_on_sc=True)


@jax.jit
def _cml(user_ids, item_ids, user_table, item_table):
    mesh = plsc.VectorSubcoreMesh(core_axis_name="c", subcore_axis_name="s")
    scan = functools.partial(
        pl.kernel,
        out_type=(jax.ShapeDtypeStruct((STAG, PADW), jnp.float32),
                  jax.ShapeDtypeStruct((STAG, PADW), jnp.float32)),
        mesh=mesh,
        compiler_params=_params,
        scratch_types=[
            pltpu.VMEM((SCHUNK,), jnp.int32),
            pltpu.VMEM((HCAP,), jnp.int32),
            pltpu.VMEM((HCAP,), jnp.int32),
            pltpu.VMEM((HCAP,), jnp.int32),
            pltpu.VMEM((HCAP,), jnp.int32),
            pltpu.VMEM_SHARED((EMBED_DIM, WINE), jnp.float32),
            pltpu.VMEM_SHARED((EMBED_DIM, WINE), jnp.float32),
            pltpu.VMEM_SHARED((EMBED_DIM, WINE), jnp.float32),
            pltpu.VMEM_SHARED((EMBED_DIM, WINE), jnp.float32),
            pltpu.VMEM((EMBED_DIM, SLICE), jnp.float32),
            pltpu.VMEM((EMBED_DIM, SLICE), jnp.float32),
            pltpu.VMEM((WCAP,), jnp.int32),
            pltpu.VMEM((WCAP,), jnp.int32),
            pltpu.VMEM((WCAP,), jnp.int32),
            pltpu.VMEM((WCAP,), jnp.int32),
            pltpu.VMEM((WCAP,), jnp.int32),
            pltpu.VMEM((WCAP,), jnp.int32),
            pltpu.VMEM((WCAP, PADW), jnp.float32),
            pltpu.VMEM((WCAP, PADW), jnp.float32),
            pltpu.VMEM((WCAP, PADW), jnp.float32),
            pltpu.VMEM((WCAP, PADW), jnp.float32),
        ] + [pltpu.SemaphoreType.DMA] * 8,
    )(_scan_body)
    ustag, istag = scan(user_ids, item_ids, user_table.T, item_table.T)

    dist = functools.partial(
        pl.kernel,
        out_type=jax.ShapeDtypeStruct((BATCH,), jnp.float32),
        mesh=mesh,
        compiler_params=_params,
        scratch_types=[
            pltpu.VMEM((HB, PADW), jnp.float32),
            pltpu.VMEM((HB, PADW), jnp.float32),
            pltpu.VMEM((BPW,), jnp.float32),
            pltpu.SemaphoreType.DMA,
            pltpu.SemaphoreType.DMA,
        ],
    )(_dist_body)
    return dist(ustag, istag)


def kernel(user_ids, item_ids, user_table, item_table):
    return _cml(user_ids, item_ids, user_table, item_table)


# speedup vs baseline: 1.7200x; 1.2568x over previous
import functools

import jax
import jax.numpy as jnp
from jax import lax
from jax.experimental import pallas as pl
from jax.experimental.pallas import tpu as pltpu
from jax.experimental.pallas import tpu_sc as plsc

NUM_LANES = 16
NUM_CORES = 2
NUM_SUBCORES = 16
NUM_WORKERS = NUM_CORES * NUM_SUBCORES

BATCH = 16384
EMBED_DIM = 32
NROWS = 1000000
PADW = 128

WINE = 1024
NWIN = 977
LASTBASE = 999040
WPW = NWIN // NUM_WORKERS
WEXTRA = NWIN - WPW * NUM_WORKERS
HCAP = 1024
WCAP = 64
NDUMP = WCAP
STAG = BATCH + NDUMP

BPW = BATCH // NUM_WORKERS
NBLK = BPW // NUM_LANES


def _win_base(w):
    return pl.multiple_of(jnp.minimum(w * WINE, LASTBASE), 128)


def _compact_hits(ids_v, he_v, hp_v, w0, w1):
    lanei = lax.iota(jnp.int32, 16)
    UNROLL = 4

    def body(v4, off):
        es, ranks, pcs, masks = [], [], [], []
        for k in range(UNROLL):
            v = v4 * UNROLL + k
            e = ids_v[pl.ds(v * NUM_LANES, NUM_LANES)]
            win = jnp.minimum(lax.shift_right_logical(e, 10), NWIN - 1)
            m = (win >= w0) & (win < w1)
            es.append(e)
            masks.append(m)
            ranks.append(plsc.cumsum(m.astype(jnp.int32)) - 1)
            pcs.append(plsc.all_reduce_population_count(m)[0])
        for k in range(UNROLL):
            v = v4 * UNROLL + k
            slots = off + ranks[k]
            plsc.store_scatter(he_v, [slots], es[k], mask=masks[k])
            pos = v * NUM_LANES + lanei
            plsc.store_scatter(hp_v, [slots], pos, mask=masks[k])
            off = off + pcs[k]
        return off

    return lax.fori_loop(0, BATCH // NUM_LANES // UNROLL, body, jnp.int32(0))


def _wave_hits(he_v, hp_v, cnt, wtarget, wcol_v, wpos_v, eb):
    lanei = lax.iota(jnp.int32, 16)
    for k in range(WCAP // NUM_LANES):
        wpos_v[pl.ds(k * NUM_LANES, NUM_LANES)] = (
            BATCH + k * NUM_LANES + lanei)

    UNROLL = 4

    def body(hv4, woff):
        es, ps, ranks, pcs, masks = [], [], [], [], []
        for k in range(UNROLL):
            base = (hv4 * UNROLL + k) * NUM_LANES
            e = he_v[pl.ds(pl.multiple_of(base, NUM_LANES), NUM_LANES)]
            p = hp_v[pl.ds(pl.multiple_of(base, NUM_LANES), NUM_LANES)]
            win = jnp.minimum(lax.shift_right_logical(e, 10), NWIN - 1)
            m = (win == wtarget) & (base + lanei < cnt)
            es.append(e)
            ps.append(p)
            masks.append(m)
            ranks.append(plsc.cumsum(m.astype(jnp.int32)) - 1)
            pcs.append(plsc.all_reduce_population_count(m)[0])
        for k in range(UNROLL):
            slots = woff + ranks[k]
            plsc.store_scatter(wcol_v, [slots], es[k] - eb, mask=masks[k])
            plsc.store_scatter(wpos_v, [slots], ps[k], mask=masks[k])
            woff = woff + pcs[k]
        return woff

    nhv4 = lax.shift_right_logical(cnt + UNROLL * NUM_LANES - 1, 6)
    return lax.fori_loop(0, nhv4, body, jnp.int32(0))


def _gather_rows(win_v, wcol_v, wcnt, row_v):
    lanei = lax.iota(jnp.int32, 16)

    def body(g, _):
        base = g * NUM_LANES
        col = wcol_v[pl.ds(pl.multiple_of(base, NUM_LANES), NUM_LANES)]
        valid = base + lanei < wcnt
        slot = base + lanei
        for d in range(EMBED_DIM):
            dvec = jnp.full((NUM_LANES,), d, jnp.int32)
            vals = plsc.load_gather(win_v, [dvec, col], mask=valid)
            plsc.store_scatter(row_v, [slot, dvec], vals, mask=valid)
        return 0

    ngv = lax.shift_right_logical(wcnt + NUM_LANES - 1, 4)
    lax.fori_loop(0, ngv, body, 0)


def _scan_body(uids_hbm, iids_hbm, utab_hbm, itab_hbm,
               ustag_hbm, istag_hbm,
               ids_v, uhe_v, uhp_v, ihe_v, ihp_v,
               uwin, iwin,
               ucol_v, upos0, upos1, icol_v, ipos0, ipos1,
               urow0, urow1, irow0, irow1,
               uwsem, iwsem, usc0, usc1, isc0, isc1):
    wid = lax.axis_index("s") * NUM_CORES + lax.axis_index("c")
    w0 = wid * WPW + jnp.minimum(wid, WEXTRA)
    nw = WPW + (wid < WEXTRA).astype(jnp.int32)

    uposs = (upos0, upos1)
    iposs = (ipos0, ipos1)
    urows = (urow0, urow1)
    irows = (irow0, irow1)
    uscs = (usc0, usc1)
    iscs = (isc0, isc1)

    def fire(t):
        eb = _win_base(w0 + t)
        for g in range(EMBED_DIM // 8):
            rs = pl.ds(8 * g, 8)
            pltpu.async_copy(utab_hbm.at[rs, pl.ds(eb, WINE)],
                             uwin.at[rs], uwsem)
            pltpu.async_copy(itab_hbm.at[rs, pl.ds(eb, WINE)],
                             iwin.at[rs], iwsem)

    def wait_win(t):
        eb = _win_base(w0 + t)
        pltpu.make_async_copy(utab_hbm.at[:, pl.ds(eb, WINE)],
                              uwin, uwsem).wait()
        pltpu.make_async_copy(itab_hbm.at[:, pl.ds(eb, WINE)],
                              iwin, iwsem).wait()

    fire(0)
    pltpu.sync_copy(uids_hbm, ids_v)
    ucnt = _compact_hits(ids_v, uhe_v, uhp_v, w0, w0 + nw)
    pltpu.sync_copy(iids_hbm, ids_v)
    icnt = _compact_hits(ids_v, ihe_v, ihp_v, w0, w0 + nw)

    def step(t, b):
        eb = _win_base(w0 + t)
        wait_win(t)
        @pl.when(t >= 2)
        def _():
            pltpu.make_async_copy(urows[b], ustag_hbm.at[uposs[b]],
                                  uscs[b]).wait()
            pltpu.make_async_copy(irows[b], istag_hbm.at[iposs[b]],
                                  iscs[b]).wait()

        uw = _wave_hits(uhe_v, uhp_v, ucnt, w0 + t, ucol_v, uposs[b], eb)
        iw = _wave_hits(ihe_v, ihp_v, icnt, w0 + t, icol_v, iposs[b], eb)
        _gather_rows(uwin, ucol_v, uw, urows[b])
        _gather_rows(iwin, icol_v, iw, irows[b])
        pltpu.async_copy(urows[b], ustag_hbm.at[uposs[b]], uscs[b])
        pltpu.async_copy(irows[b], istag_hbm.at[iposs[b]], iscs[b])

        @pl.when(t + 1 < nw)
        def _():
            fire(t + 1)

    def outer(t2, _):
        for b in range(2):
            t = t2 * 2 + b

            @pl.when(t < nw)
            def _():
                step(t, b)
        return 0

    lax.fori_loop(0, (WPW + 2) // 2, outer, 0)

    def tail(t2, _):
        for b in range(2):
            t = t2 * 2 + b

            @pl.when((t < nw) & (t + 2 >= nw))
            def _():
                pltpu.make_async_copy(urows[b], ustag_hbm.at[uposs[b]],
                                      uscs[b]).wait()
                pltpu.make_async_copy(irows[b], istag_hbm.at[iposs[b]],
                                      iscs[b]).wait()
        return 0

    lax.fori_loop(0, (WPW + 2) // 2, tail, 0)


HB = BPW // 2


def _dist_body(ustag_hbm, istag_hbm, out_hbm, ubuf_v, ibuf_v, out_v,
               usem, isem):
    wid = lax.axis_index("s") * NUM_CORES + lax.axis_index("c")
    base = wid * BPW

    lane = lax.iota(jnp.int32, 16)
    zero = jnp.zeros((NUM_LANES,), jnp.float32)
    half = EMBED_DIM // 2

    def load_half(h):
        off = pl.multiple_of(base + h * HB, HB)
        cu = pltpu.async_copy(ustag_hbm.at[pl.ds(off, HB)], ubuf_v, usem)
        ci = pltpu.async_copy(istag_hbm.at[pl.ds(off, HB)], ibuf_v, isem)
        cu.wait()
        ci.wait()

    def blk(blk_i, _):
        h = blk_i // (HB // NUM_LANES)

        @pl.when((blk_i % (HB // NUM_LANES)) == 0)
        def _():
            load_half(h)

        base_row = pl.multiple_of(
            (blk_i % (HB // NUM_LANES)) * NUM_LANES, NUM_LANES)
        acc_p, acc_u, acc_i = zero, zero, zero
        for r in range(NUM_LANES):
            row = base_row + r
            u0 = ubuf_v[row, pl.ds(0, half)]
            u1 = ubuf_v[row, pl.ds(half, half)]
            i0 = ibuf_v[row, pl.ds(0, half)]
            i1 = ibuf_v[row, pl.ds(half, half)]
            p0 = u0 * i0
            p1 = u1 * i1
            sp = jnp.sum(p0 * p0 + p1 * p1)
            su = jnp.sum(u0 * u0 + u1 * u1)
            si = jnp.sum(i0 * i0 + i1 * i1)
            m = lane == r
            acc_p = jnp.where(m, sp, acc_p)
            acc_u = jnp.where(m, su, acc_u)
            acc_i = jnp.where(m, si, acc_i)
        denom = jnp.maximum(acc_u, 1.0) * jnp.maximum(acc_i, 1.0)
        out_v[pl.ds(pl.multiple_of(h * HB, HB) + base_row, NUM_LANES)] = (
            -(acc_p / denom))
        return 0

    lax.fori_loop(0, NBLK, blk, 0)
    pltpu.sync_copy(out_v, out_hbm.at[pl.ds(base, BPW)])


_params = pltpu.CompilerParams(needs_layout_passes=False,
                               use_tc_tiling_on_sc=True)


@jax.jit
def _cml(user_ids, item_ids, user_table, item_table):
    mesh = plsc.VectorSubcoreMesh(core_axis_name="c", subcore_axis_name="s")
    scan = functools.partial(
        pl.kernel,
        out_type=(jax.ShapeDtypeStruct((STAG, PADW), jnp.float32),
                  jax.ShapeDtypeStruct((STAG, PADW), jnp.float32)),
        mesh=mesh,
        compiler_params=_params,
        scratch_types=[
            pltpu.VMEM((BATCH,), jnp.int32),
            pltpu.VMEM((HCAP,), jnp.int32),
            pltpu.VMEM((HCAP,), jnp.int32),
            pltpu.VMEM((HCAP,), jnp.int32),
            pltpu.VMEM((HCAP,), jnp.int32),
            pltpu.VMEM((EMBED_DIM, WINE), jnp.float32),
            pltpu.VMEM((EMBED_DIM, WINE), jnp.float32),
            pltpu.VMEM((WCAP,), jnp.int32),
            pltpu.VMEM((WCAP,), jnp.int32),
            pltpu.VMEM((WCAP,), jnp.int32),
            pltpu.VMEM((WCAP,), jnp.int32),
            pltpu.VMEM((WCAP,), jnp.int32),
            pltpu.VMEM((WCAP,), jnp.int32),
            pltpu.VMEM((WCAP, PADW), jnp.float32),
            pltpu.VMEM((WCAP, PADW), jnp.float32),
            pltpu.VMEM((WCAP, PADW), jnp.float32),
            pltpu.VMEM((WCAP, PADW), jnp.float32),
        ] + [pltpu.SemaphoreType.DMA] * 6,
    )(_scan_body)
    ustag, istag = scan(user_ids, item_ids, user_table.T, item_table.T)

    dist = functools.partial(
        pl.kernel,
        out_type=jax.ShapeDtypeStruct((BATCH,), jnp.float32),
        mesh=mesh,
        compiler_params=_params,
        scratch_types=[
            pltpu.VMEM((HB, PADW), jnp.float32),
            pltpu.VMEM((HB, PADW), jnp.float32),
            pltpu.VMEM((BPW,), jnp.float32),
            pltpu.SemaphoreType.DMA,
            pltpu.SemaphoreType.DMA,
        ],
    )(_dist_body)
    return dist(ustag, istag)


def kernel(user_ids, item_ids, user_table, item_table):
    return _cml(user_ids, item_ids, user_table, item_table)
